# baseline (speedup 1.0000x reference)
import jax
import jax.numpy as jnp
from jax.experimental import pallas as pl
from jax.experimental.pallas import tpu as pltpu

NUM_CLASSES = 81
EPS = 1e-3

N = 5000
P = 7
C = 256
H = 1024

BM = 1000
NM = N // BM


def _head_kernel(x_ref, w1_ref, s1_ref, t1_ref, w2_ref, s2_ref, t2_ref,
                 wc_ref, bc_ref, wo_ref, bo_ref,
                 logit_ref, prob_ref, off_ref, acc_ref):
    k = pl.program_id(1)

    psum = jnp.dot(x_ref[0].astype(jnp.bfloat16), w1_ref[k, 0],
                   preferred_element_type=jnp.float32)
    for w in range(1, P):
        psum += jnp.dot(x_ref[w].astype(jnp.bfloat16), w1_ref[k, w],
                        preferred_element_type=jnp.float32)

    @pl.when(k == 0)
    def _init():
        acc_ref[...] = psum

    @pl.when(k > 0)
    def _accum():
        acc_ref[...] += psum

    @pl.when(k == P - 1)
    def _epilogue():
        y1 = jnp.maximum(acc_ref[...] * s1_ref[...] + t1_ref[...], 0.0)
        y2 = jnp.dot(y1.astype(jnp.bfloat16), w2_ref[...],
                     preferred_element_type=jnp.float32)
        y2 = jnp.maximum(y2 * s2_ref[...] + t2_ref[...], 0.0)
        y2b = y2.astype(jnp.bfloat16)
        logits = jnp.dot(y2b, wc_ref[...],
                         preferred_element_type=jnp.float32) + bc_ref[...]
        logit_ref[...] = logits
        mx = jnp.max(logits, axis=-1, keepdims=True)
        e = jnp.exp(logits - mx)
        prob_ref[...] = e / jnp.sum(e, axis=-1, keepdims=True)
        off_ref[...] = jnp.dot(y2b, wo_ref[...],
                               preferred_element_type=jnp.float32) + bo_ref[...]


def kernel(inputs, W1, b1, g1, be1, m1, v1, W2, b2, g2, be2, m2, v2, Wc, bc, Wo, bo):
    xt = inputs.transpose(1, 2, 0, 3)

    w1b = W1.astype(jnp.bfloat16)
    w2b = W2.reshape(H, H).astype(jnp.bfloat16)
    wcb = Wc.astype(jnp.bfloat16)
    wob = Wo.astype(jnp.bfloat16)

    s1 = g1 * jax.lax.rsqrt(v1 + EPS)
    t1 = s1 * (b1 - m1) + be1
    s2 = g2 * jax.lax.rsqrt(v2 + EPS)
    t2 = s2 * (b2 - m2) + be2

    const = lambda bs: pl.BlockSpec(bs, lambda m, k: (0,) * len(bs))

    logit, prob, off = pl.pallas_call(
        _head_kernel,
        grid=(NM, P),
        in_specs=[
            pl.BlockSpec((None, P, BM, C), lambda m, k: (k, 0, m, 0)),
            const((P, P, C, H)),
            const((1, H)), const((1, H)),
            const((H, H)),
            const((1, H)), const((1, H)),
            const((H, NUM_CLASSES)), const((1, NUM_CLASSES)),
            const((H, 4 * NUM_CLASSES)), const((1, 4 * NUM_CLASSES)),
        ],
        out_specs=[
            pl.BlockSpec((BM, NUM_CLASSES), lambda m, k: (m, 0)),
            pl.BlockSpec((BM, NUM_CLASSES), lambda m, k: (m, 0)),
            pl.BlockSpec((BM, 4 * NUM_CLASSES), lambda m, k: (m, 0)),
        ],
        out_shape=[
            jax.ShapeDtypeStruct((N, NUM_CLASSES), jnp.float32),
            jax.ShapeDtypeStruct((N, NUM_CLASSES), jnp.float32),
            jax.ShapeDtypeStruct((N, 4 * NUM_CLASSES), jnp.float32),
        ],
        scratch_shapes=[pltpu.VMEM((BM, H), jnp.float32)],
        compiler_params=pltpu.CompilerParams(
            dimension_semantics=("arbitrary", "arbitrary"),
        ),
    )(xt, w1b,
      s1.reshape(1, H), t1.reshape(1, H),
      w2b,
      s2.reshape(1, H), t2.reshape(1, H),
      wcb, bc.reshape(1, NUM_CLASSES),
      wob, bo.reshape(1, 4 * NUM_CLASSES))

    return logit, prob, off.reshape(N, NUM_CLASSES, 4)


# speedup vs baseline: 1.0095x; 1.0095x over previous
import jax
import jax.numpy as jnp
from jax.experimental import pallas as pl
from jax.experimental.pallas import tpu as pltpu

NUM_CLASSES = 81
EPS = 1e-3

N = 5000
P = 7
C = 256
H = 1024
K = P * C

BM = 1000
NM = N // BM


def _head_kernel(x_ref, w1_ref, s1_ref, t1_ref, w2_ref, s2_ref, t2_ref,
                 wc_ref, bc_ref, wo_ref, bo_ref,
                 logit_ref, prob_ref, off_ref, xcat_ref, acc_ref):
    k = pl.program_id(1)

    for w in range(P):
        xcat_ref[:, w * C:(w + 1) * C] = x_ref[w].astype(jnp.bfloat16)
    psum = jnp.dot(xcat_ref[...], w1_ref[k],
                   preferred_element_type=jnp.float32)

    @pl.when(k == 0)
    def _init():
        acc_ref[...] = psum

    @pl.when(k > 0)
    def _accum():
        acc_ref[...] += psum

    @pl.when(k == P - 1)
    def _epilogue():
        y1 = jnp.maximum(acc_ref[...] * s1_ref[...] + t1_ref[...], 0.0)
        y2 = jnp.dot(y1.astype(jnp.bfloat16), w2_ref[...],
                     preferred_element_type=jnp.float32)
        y2 = jnp.maximum(y2 * s2_ref[...] + t2_ref[...], 0.0)
        y2b = y2.astype(jnp.bfloat16)
        logits = jnp.dot(y2b, wc_ref[...],
                         preferred_element_type=jnp.float32) + bc_ref[...]
        logit_ref[...] = logits
        mx = jnp.max(logits, axis=-1, keepdims=True)
        e = jnp.exp(logits - mx)
        prob_ref[...] = e / jnp.sum(e, axis=-1, keepdims=True)
        off_ref[...] = jnp.dot(y2b, wo_ref[...],
                               preferred_element_type=jnp.float32) + bo_ref[...]


def kernel(inputs, W1, b1, g1, be1, m1, v1, W2, b2, g2, be2, m2, v2, Wc, bc, Wo, bo):
    xt = inputs.transpose(1, 2, 0, 3)

    w1b = W1.astype(jnp.bfloat16).reshape(P, K, H)
    w2b = W2.reshape(H, H).astype(jnp.bfloat16)
    wcb = Wc.astype(jnp.bfloat16)
    wob = Wo.astype(jnp.bfloat16)

    s1 = g1 * jax.lax.rsqrt(v1 + EPS)
    t1 = s1 * (b1 - m1) + be1
    s2 = g2 * jax.lax.rsqrt(v2 + EPS)
    t2 = s2 * (b2 - m2) + be2

    const = lambda bs: pl.BlockSpec(bs, lambda m, k: (0,) * len(bs))

    logit, prob, off = pl.pallas_call(
        _head_kernel,
        grid=(NM, P),
        in_specs=[
            pl.BlockSpec((None, P, BM, C), lambda m, k: (k, 0, m, 0)),
            const((P, K, H)),
            const((1, H)), const((1, H)),
            const((H, H)),
            const((1, H)), const((1, H)),
            const((H, NUM_CLASSES)), const((1, NUM_CLASSES)),
            const((H, 4 * NUM_CLASSES)), const((1, 4 * NUM_CLASSES)),
        ],
        out_specs=[
            pl.BlockSpec((BM, NUM_CLASSES), lambda m, k: (m, 0)),
            pl.BlockSpec((BM, NUM_CLASSES), lambda m, k: (m, 0)),
            pl.BlockSpec((BM, 4 * NUM_CLASSES), lambda m, k: (m, 0)),
        ],
        out_shape=[
            jax.ShapeDtypeStruct((N, NUM_CLASSES), jnp.float32),
            jax.ShapeDtypeStruct((N, NUM_CLASSES), jnp.float32),
            jax.ShapeDtypeStruct((N, 4 * NUM_CLASSES), jnp.float32),
        ],
        scratch_shapes=[
            pltpu.VMEM((BM, K), jnp.bfloat16),
            pltpu.VMEM((BM, H), jnp.float32),
        ],
        compiler_params=pltpu.CompilerParams(
            dimension_semantics=("arbitrary", "arbitrary"),
        ),
    )(xt, w1b,
      s1.reshape(1, H), t1.reshape(1, H),
      w2b,
      s2.reshape(1, H), t2.reshape(1, H),
      wcb, bc.reshape(1, NUM_CLASSES),
      wob, bo.reshape(1, 4 * NUM_CLASSES))

    return logit, prob, off.reshape(N, NUM_CLASSES, 4)


# speedup vs baseline: 1.0657x; 1.0557x over previous
import jax
import jax.numpy as jnp
from jax.experimental import pallas as pl
from jax.experimental.pallas import tpu as pltpu

NUM_CLASSES = 81
EPS = 1e-3

N = 5000
P = 7
C = 256
H = 1024

BM = 1000
NM = N // BM


def _head_kernel(x_ref, w1_ref, s1_ref, t1_ref, w2_ref, s2_ref, t2_ref,
                 wc_ref, bc_ref, wo_ref, bo_ref,
                 logit_ref, prob_ref, off_ref, acc_ref):
    k = pl.program_id(1)

    psum = jnp.dot(x_ref[0].astype(jnp.bfloat16),
                   w1_ref[0].astype(jnp.bfloat16),
                   preferred_element_type=jnp.float32)
    for w in range(1, P):
        psum += jnp.dot(x_ref[w].astype(jnp.bfloat16),
                        w1_ref[w].astype(jnp.bfloat16),
                        preferred_element_type=jnp.float32)

    @pl.when(k == 0)
    def _init():
        acc_ref[...] = psum

    @pl.when(k > 0)
    def _accum():
        acc_ref[...] += psum

    @pl.when(k == P - 1)
    def _epilogue():
        y1 = jnp.maximum(acc_ref[...] * s1_ref[...] + t1_ref[...], 0.0)
        y2 = jnp.dot(y1.astype(jnp.bfloat16),
                     w2_ref[...].astype(jnp.bfloat16),
                     preferred_element_type=jnp.float32)
        y2 = jnp.maximum(y2 * s2_ref[...] + t2_ref[...], 0.0)
        y2b = y2.astype(jnp.bfloat16)
        logits = jnp.dot(y2b, wc_ref[...].astype(jnp.bfloat16),
                         preferred_element_type=jnp.float32) + bc_ref[...]
        logit_ref[...] = logits
        mx = jnp.max(logits, axis=-1, keepdims=True)
        e = jnp.exp(logits - mx)
        prob_ref[...] = e / jnp.sum(e, axis=-1, keepdims=True)
        off_ref[...] = jnp.dot(y2b, wo_ref[...].astype(jnp.bfloat16),
                               preferred_element_type=jnp.float32) + bo_ref[...]


def kernel(inputs, W1, b1, g1, be1, m1, v1, W2, b2, g2, be2, m2, v2, Wc, bc, Wo, bo):
    xt = inputs.transpose(1, 2, 0, 3)

    s1 = g1 * jax.lax.rsqrt(v1 + EPS)
    t1 = s1 * (b1 - m1) + be1
    s2 = g2 * jax.lax.rsqrt(v2 + EPS)
    t2 = s2 * (b2 - m2) + be2

    const = lambda bs: pl.BlockSpec(bs, lambda m, k: (0,) * len(bs))

    logit, prob, off = pl.pallas_call(
        _head_kernel,
        grid=(NM, P),
        in_specs=[
            pl.BlockSpec((None, P, BM, C), lambda m, k: (k, 0, m, 0)),
            pl.BlockSpec((None, P, C, H), lambda m, k: (k, 0, 0, 0)),
            const((1, H)), const((1, H)),
            pl.BlockSpec((None, None, H, H), lambda m, k: (0, 0, 0, 0)),
            const((1, H)), const((1, H)),
            const((H, NUM_CLASSES)), const((1, NUM_CLASSES)),
            const((H, 4 * NUM_CLASSES)), const((1, 4 * NUM_CLASSES)),
        ],
        out_specs=[
            pl.BlockSpec((BM, NUM_CLASSES), lambda m, k: (m, 0)),
            pl.BlockSpec((BM, NUM_CLASSES), lambda m, k: (m, 0)),
            pl.BlockSpec((BM, 4 * NUM_CLASSES), lambda m, k: (m, 0)),
        ],
        out_shape=[
            jax.ShapeDtypeStruct((N, NUM_CLASSES), jnp.float32),
            jax.ShapeDtypeStruct((N, NUM_CLASSES), jnp.float32),
            jax.ShapeDtypeStruct((N, 4 * NUM_CLASSES), jnp.float32),
        ],
        scratch_shapes=[pltpu.VMEM((BM, H), jnp.float32)],
        compiler_params=pltpu.CompilerParams(
            dimension_semantics=("arbitrary", "arbitrary"),
        ),
    )(xt, W1,
      s1.reshape(1, H), t1.reshape(1, H),
      W2,
      s2.reshape(1, H), t2.reshape(1, H),
      Wc, bc.reshape(1, NUM_CLASSES),
      Wo, bo.reshape(1, 4 * NUM_CLASSES))

    return logit, prob, off.reshape(N, NUM_CLASSES, 4)
